# baseline (speedup 1.0000x reference)
import functools

import jax
import jax.numpy as jnp
import numpy as np
from jax import lax
from jax.experimental import pallas as pl
from jax.experimental.pallas import tpu as pltpu
from jax.experimental.pallas import tpu_sc as plsc

_B = 8
_S = 1048576
_QS = _S // 4
_W = 32768
_NW = _QS // _W
_U = 8
_NB1, _NB2, _NB3 = 2048, 2048, 1024
_MINT = np.int32(-2147483648)


def _ukey(x):
    b = plsc.bitcast(x, jnp.int32)
    return b ^ (lax.shift_right_arithmetic(b, 31) | _MINT)


_HP = _NB1 + 1
_CAP = 8192


def _sc_threshold_body(pred_hbm, r_hbm, out_hbm, wina, winb, hist, flat,
                       gbuf, cand, rv, tstage, sema, semb, sh1, sh2, sh3):
    c = lax.axis_index("c")
    s = lax.axis_index("s")
    row = c * 4 + s // 4
    q = s % 4
    g = s // 4
    lane = lax.iota(jnp.int32, 16)
    laneoff = lane * _HP
    zeros16 = jnp.zeros((16,), jnp.int32)
    ones16 = jnp.ones((16,), jnp.int32)
    bufs = (wina, winb)
    sems = (sema, semb)

    pltpu.sync_copy(r_hbm, rv)
    r = rv[pl.ds(0, 16)][0]

    def zero_hist():
        @plsc.parallel_loop(0, 16 * _HP + 16, 16, unroll=8)
        def _(i):
            hist[pl.ds(i, 16)] = zeros16

    def src(w):
        return pred_hbm.at[row, pl.ds(q * _QS + w * _W, _W)]

    def full_scan(bucket_and_mask):
        def hist_window(win):
            @plsc.parallel_loop(0, _W, 16, unroll=_U)
            def _(i):
                x = win[pl.ds(i, 16)]
                uk = _ukey(x)
                bucket, mask = bucket_and_mask(uk)
                addr = laneoff + bucket
                if mask is None:
                    plsc.addupdate_scatter(hist, [addr], ones16)
                else:
                    plsc.addupdate_scatter(hist, [addr], ones16, mask=mask)

        copies = [pltpu.async_copy(src(0), bufs[0], sems[0]), None]
        for w in range(_NW):
            if w + 1 < _NW:
                copies[(w + 1) % 2] = pltpu.async_copy(
                    src(w + 1), bufs[(w + 1) % 2], sems[(w + 1) % 2])
            copies[w % 2].wait()
            hist_window(bufs[w % 2])

    def finish_pass(nb, shp, r):
        @plsc.parallel_loop(0, nb, 16, unroll=2)
        def _(o):
            acc = hist[pl.ds(o, 16)]
            for l in range(1, 16):
                acc = acc + hist[pl.ds(l * _HP + o, 16)]
            flat[pl.ds(o, 16)] = acc

        pltpu.sync_copy(flat.at[pl.ds(0, _NB1)], shp.at[s])
        plsc.subcore_barrier()
        pltpu.sync_copy(shp.at[pl.ds(g * 4, 4)], gbuf)

        def gsum(j, _):
            o = j * 16
            flat[pl.ds(o, 16)] = (
                gbuf[0, pl.ds(o, 16)] + gbuf[1, pl.ds(o, 16)]
                + gbuf[2, pl.ds(o, 16)] + gbuf[3, pl.ds(o, 16)])
            return 0
        lax.fori_loop(0, nb // 16, gsum, 0)

        rvec = jnp.broadcast_to(r, (16,))

        def ploop(j, carry):
            run, cnt, exm = carry
            v = flat[pl.ds(j * 16, 16)]
            cum = plsc.cumsum(v) + run
            m = cum <= rvec
            cnt = cnt + jnp.where(m, 1, 0)
            exm = jnp.maximum(exm, jnp.where(m, cum, 0))
            run = jnp.broadcast_to(jnp.max(cum), (16,))
            return run, cnt, exm
        _, cnt, exm = lax.fori_loop(0, nb // 16, ploop,
                                    (zeros16, zeros16, zeros16))
        return jnp.sum(cnt), r - jnp.max(exm)

    zero_hist()
    full_scan(lambda uk: (lax.shift_right_logical(uk, 21), None))
    b1, r = finish_pass(_NB1, sh1, r)
    b1v = jnp.broadcast_to(b1, (16,))

    zero_hist()
    capv = jnp.full((16,), _CAP - 16, jnp.int32)

    def hist2_window(win, cnt_v):
        @plsc.parallel_loop(0, _W, 16, unroll=_U, carry=cnt_v)
        def body(i, cnt_v):
            x = win[pl.ds(i, 16)]
            uk = _ukey(x)
            match = lax.shift_right_logical(uk, 21) == b1v
            bucket = lax.shift_right_logical(uk, 10) & jnp.int32(0x7FF)
            plsc.addupdate_scatter(hist, [laneoff + bucket], ones16,
                                   mask=match)
            smask = jnp.logical_and(match, cnt_v <= capv)
            plsc.store_compressed(cand.at[pl.ds(cnt_v[0], 16)], x,
                                  mask=smask)
            return cnt_v + plsc.all_reduce_population_count(match)
        return body

    cnt_v = zeros16
    copies = [pltpu.async_copy(src(0), bufs[0], sems[0]), None]
    for w in range(_NW):
        if w + 1 < _NW:
            copies[(w + 1) % 2] = pltpu.async_copy(
                src(w + 1), bufs[(w + 1) % 2], sems[(w + 1) % 2])
        copies[w % 2].wait()
        cnt_v = hist2_window(bufs[w % 2], cnt_v)
    b2, r = finish_pass(_NB2, sh2, r)
    p2v = jnp.broadcast_to(b1 * 2048 + b2, (16,))

    zero_hist()
    ncand = cnt_v[0]

    @pl.when(ncand <= _CAP - 16)
    def _():
        def cbody(i, _):
            o = i * 16
            x = cand[pl.ds(o, 16)]
            uk = _ukey(x)
            valid = (o + lane) < jnp.broadcast_to(ncand, (16,))
            m3 = jnp.logical_and(
                valid, lax.shift_right_logical(uk, 10) == p2v)
            plsc.addupdate_scatter(
                hist, [laneoff + (uk & jnp.int32(0x3FF))], ones16, mask=m3)
            return 0
        lax.fori_loop(0, (ncand + 15) // 16, cbody, 0)

    @pl.when(ncand > _CAP - 16)
    def _():
        full_scan(lambda uk: (uk & jnp.int32(0x3FF),
                              lax.shift_right_logical(uk, 10) == p2v))

    b3, _ = finish_pass(_NB3, sh3, r)

    ukey = b1 * 2097152 + b2 * 1024 + b3
    key = ukey ^ _MINT
    bits = key ^ (lax.shift_right_arithmetic(key, 31)
                  & jnp.int32(0x7FFFFFFF))
    thr = plsc.bitcast(jnp.broadcast_to(bits, (16,)), jnp.float32)

    @pl.when(q == 0)
    def _():
        tstage[pl.ds(0, 16)] = thr
        pltpu.sync_copy(tstage, out_hbm.at[row])


def _sc_threshold(pred, r_arr):
    mesh = plsc.VectorSubcoreMesh(core_axis_name="c", subcore_axis_name="s")
    return pl.kernel(
        _sc_threshold_body,
        out_type=jax.ShapeDtypeStruct((_B, 16), jnp.float32),
        mesh=mesh,
        compiler_params=pltpu.CompilerParams(needs_layout_passes=False),
        scratch_types=[
            pltpu.VMEM((_W,), jnp.float32),
            pltpu.VMEM((_W,), jnp.float32),
            pltpu.VMEM((16 * _HP + 16,), jnp.int32),
            pltpu.VMEM((_NB1 + 16,), jnp.int32),
            pltpu.VMEM((4, _NB1), jnp.int32),
            pltpu.VMEM((_CAP + 16,), jnp.float32),
            pltpu.VMEM((16,), jnp.int32),
            pltpu.VMEM((16,), jnp.float32),
            pltpu.SemaphoreType.DMA,
            pltpu.SemaphoreType.DMA,
            pltpu.VMEM_SHARED((16, _NB1), jnp.int32),
            pltpu.VMEM_SHARED((16, _NB1), jnp.int32),
            pltpu.VMEM_SHARED((16, _NB1), jnp.int32),
        ],
    )(pred, r_arr)


def _mask_body(pred_ref, thr_ref, keep_ref, pruned_ref):
    x = pred_ref[...]
    m = x > thr_ref[:, 0:1]
    keep_ref[...] = m
    pruned_ref[...] = jnp.where(m, x, jnp.float32(0.0))


def _masked_outputs(pred, thr, interpret=False):
    B, S = pred.shape
    blk = min(S, 65536)
    return pl.pallas_call(
        _mask_body,
        grid=(S // blk,),
        in_specs=[
            pl.BlockSpec((B, blk), lambda j: (0, j)),
            pl.BlockSpec((B, thr.shape[1]), lambda j: (0, 0)),
        ],
        out_specs=[
            pl.BlockSpec((B, blk), lambda j: (0, j)),
            pl.BlockSpec((B, blk), lambda j: (0, j)),
        ],
        out_shape=[
            jax.ShapeDtypeStruct((B, S), jnp.bool_),
            jax.ShapeDtypeStruct((B, S), jnp.float32),
        ],
        interpret=interpret,
    )(pred, thr)


@jax.jit
def _run(pred, k):
    B, S = pred.shape
    r = jnp.int32(S - 1) - k.astype(jnp.int32)
    thr = _sc_threshold(pred, jnp.full((16,), r, jnp.int32))
    return _masked_outputs(pred, thr)


def kernel(pred, k):
    return _run(pred, jnp.asarray(k))


# speedup vs baseline: 1.0436x; 1.0436x over previous
import functools

import jax
import jax.numpy as jnp
import numpy as np
from jax import lax
from jax.experimental import pallas as pl
from jax.experimental.pallas import tpu as pltpu
from jax.experimental.pallas import tpu_sc as plsc

_B = 8
_S = 1048576
_QS = _S // 4
_W = 32768
_NW = _QS // _W
_U = 8
_NB1, _NB2, _NB3 = 2048, 2048, 1024
_MINT = np.int32(-2147483648)


def _ukey(x):
    b = plsc.bitcast(x, jnp.int32)
    return b ^ (lax.shift_right_arithmetic(b, 31) | _MINT)


_HP = _NB1 + 1


def _sc_threshold_body(pred_hbm, r_hbm, out_hbm, wina, winb, hist, flat,
                       gbuf, rv, tstage, sema, semb, sh1, sh2, sh3):
    c = lax.axis_index("c")
    s = lax.axis_index("s")
    row = c * 4 + s // 4
    q = s % 4
    g = s // 4
    lane = lax.iota(jnp.int32, 16)
    laneoff = lane * _HP
    zeros16 = jnp.zeros((16,), jnp.int32)
    ones16 = jnp.ones((16,), jnp.int32)
    bufs = (wina, winb)
    sems = (sema, semb)

    pltpu.sync_copy(r_hbm, rv)
    r = rv[pl.ds(0, 16)][0]

    def zero_hist():
        @plsc.parallel_loop(0, 16 * _HP + 16, 16, unroll=8)
        def _(i):
            hist[pl.ds(i, 16)] = zeros16

    def src(w):
        return pred_hbm.at[row, pl.ds(q * _QS + w * _W, _W)]

    def full_scan(bucket_and_mask):
        def hist_window(win):
            @plsc.parallel_loop(0, _W, 16, unroll=_U)
            def _(i):
                x = win[pl.ds(i, 16)]
                uk = _ukey(x)
                bucket, mask = bucket_and_mask(uk)
                addr = laneoff + bucket
                if mask is None:
                    plsc.addupdate_scatter(hist, [addr], ones16)
                else:
                    plsc.addupdate_scatter(hist, [addr], ones16, mask=mask)

        copies = [pltpu.async_copy(src(0), bufs[0], sems[0]), None]
        for w in range(_NW):
            if w + 1 < _NW:
                copies[(w + 1) % 2] = pltpu.async_copy(
                    src(w + 1), bufs[(w + 1) % 2], sems[(w + 1) % 2])
            copies[w % 2].wait()
            hist_window(bufs[w % 2])

    def finish_pass(nb, shp, r):
        @plsc.parallel_loop(0, nb, 16, unroll=2)
        def _(o):
            acc = hist[pl.ds(o, 16)]
            for l in range(1, 16):
                acc = acc + hist[pl.ds(l * _HP + o, 16)]
            flat[pl.ds(o, 16)] = acc

        pltpu.sync_copy(flat.at[pl.ds(0, _NB1)], shp.at[s])
        plsc.subcore_barrier()
        pltpu.sync_copy(shp.at[pl.ds(g * 4, 4)], gbuf)

        def gsum(j, _):
            o = j * 16
            flat[pl.ds(o, 16)] = (
                gbuf[0, pl.ds(o, 16)] + gbuf[1, pl.ds(o, 16)]
                + gbuf[2, pl.ds(o, 16)] + gbuf[3, pl.ds(o, 16)])
            return 0
        lax.fori_loop(0, nb // 16, gsum, 0)

        rvec = jnp.broadcast_to(r, (16,))

        def ploop(j, carry):
            run, cnt, exm = carry
            v = flat[pl.ds(j * 16, 16)]
            cum = plsc.cumsum(v) + run
            m = cum <= rvec
            cnt = cnt + jnp.where(m, 1, 0)
            exm = jnp.maximum(exm, jnp.where(m, cum, 0))
            run = jnp.broadcast_to(jnp.max(cum), (16,))
            return run, cnt, exm
        _, cnt, exm = lax.fori_loop(0, nb // 16, ploop,
                                    (zeros16, zeros16, zeros16))
        return jnp.sum(cnt), r - jnp.max(exm)

    zero_hist()
    full_scan(lambda uk: (lax.shift_right_logical(uk, 21), None))
    b1, r = finish_pass(_NB1, sh1, r)
    b1v = jnp.broadcast_to(b1, (16,))

    zero_hist()
    full_scan(lambda uk: (lax.shift_right_logical(uk, 10) & jnp.int32(0x7FF),
                          lax.shift_right_logical(uk, 21) == b1v))
    b2, r = finish_pass(_NB2, sh2, r)
    p2v = jnp.broadcast_to(b1 * 2048 + b2, (16,))

    zero_hist()
    full_scan(lambda uk: (uk & jnp.int32(0x3FF),
                          lax.shift_right_logical(uk, 10) == p2v))
    b3, _ = finish_pass(_NB3, sh3, r)

    ukey = b1 * 2097152 + b2 * 1024 + b3
    key = ukey ^ _MINT
    bits = key ^ (lax.shift_right_arithmetic(key, 31)
                  & jnp.int32(0x7FFFFFFF))
    thr = plsc.bitcast(jnp.broadcast_to(bits, (16,)), jnp.float32)

    @pl.when(q == 0)
    def _():
        tstage[pl.ds(0, 16)] = thr
        pltpu.sync_copy(tstage, out_hbm.at[row])


def _sc_threshold(pred, r_arr):
    mesh = plsc.VectorSubcoreMesh(core_axis_name="c", subcore_axis_name="s")
    return pl.kernel(
        _sc_threshold_body,
        out_type=jax.ShapeDtypeStruct((_B, 16), jnp.float32),
        mesh=mesh,
        compiler_params=pltpu.CompilerParams(needs_layout_passes=False),
        scratch_types=[
            pltpu.VMEM((_W,), jnp.float32),
            pltpu.VMEM((_W,), jnp.float32),
            pltpu.VMEM((16 * _HP + 16,), jnp.int32),
            pltpu.VMEM((_NB1 + 16,), jnp.int32),
            pltpu.VMEM((4, _NB1), jnp.int32),
            pltpu.VMEM((16,), jnp.int32),
            pltpu.VMEM((16,), jnp.float32),
            pltpu.SemaphoreType.DMA,
            pltpu.SemaphoreType.DMA,
            pltpu.VMEM_SHARED((16, _NB1), jnp.int32),
            pltpu.VMEM_SHARED((16, _NB1), jnp.int32),
            pltpu.VMEM_SHARED((16, _NB1), jnp.int32),
        ],
    )(pred, r_arr)


def _mask_body(pred_ref, thr_ref, keep_ref, pruned_ref):
    x = pred_ref[...]
    m = x > thr_ref[:, 0:1]
    keep_ref[...] = m
    pruned_ref[...] = jnp.where(m, x, jnp.float32(0.0))


def _masked_outputs(pred, thr, interpret=False):
    B, S = pred.shape
    blk = min(S, 131072)
    return pl.pallas_call(
        _mask_body,
        grid=(S // blk,),
        in_specs=[
            pl.BlockSpec((B, blk), lambda j: (0, j)),
            pl.BlockSpec((B, thr.shape[1]), lambda j: (0, 0)),
        ],
        out_specs=[
            pl.BlockSpec((B, blk), lambda j: (0, j)),
            pl.BlockSpec((B, blk), lambda j: (0, j)),
        ],
        out_shape=[
            jax.ShapeDtypeStruct((B, S), jnp.bool_),
            jax.ShapeDtypeStruct((B, S), jnp.float32),
        ],
        interpret=interpret,
    )(pred, thr)


@jax.jit
def _run(pred, k):
    B, S = pred.shape
    r = jnp.int32(S - 1) - k.astype(jnp.int32)
    thr = _sc_threshold(pred, jnp.full((16,), r, jnp.int32))
    return _masked_outputs(pred, thr)


def kernel(pred, k):
    return _run(pred, jnp.asarray(k))
